# Initial kernel scaffold; baseline (speedup 1.0000x reference)
import jax
import jax.numpy as jnp
from jax.experimental import pallas as pl


def kernel(at_no, pos, edge_index, batch, embed, W_rbf, b_rbf, Wm1, bm1, Wm2, bm2, WU, WV, Wa1, ba1, Wa2, ba2, Wo1, bo1, Wo2, bo2):
    raise NotImplementedError("write your pallas kernel here")


# speedup vs baseline: 6.2250x; 6.2250x over previous
import functools

import jax
import jax.numpy as jnp
from jax import lax
from jax.experimental import pallas as pl
from jax.experimental.pallas import tpu as pltpu
from jax.experimental.pallas import tpu_sc as plsc

N = 10000
E = 160000
D = 256
NB = 20
NBLK = 3
NG = 16
H = 512
CUT = 5.0

NPAD = 10240
EPAD = 160768
TW = 40
PASSES = 8
NOUT = 32 * TW * PASSES
RPLEN = NOUT + 16
CE = 32

_mesh = plsc.VectorSubcoreMesh(
    core_axis_name="c", subcore_axis_name="s", num_cores=2, num_subcores=16)


def _sc_gather(table, idx, chunk):
    B = idx.shape[0]
    R = table.shape[1]
    bpw = B // 32
    nch = -(-bpw // chunk)

    @functools.partial(
        pl.kernel,
        out_type=jax.ShapeDtypeStruct((B, R), jnp.float32),
        mesh=_mesh,
        scratch_types=[
            pltpu.VMEM((chunk,), jnp.int32),
            pltpu.VMEM((chunk, R), jnp.float32),
            pltpu.SemaphoreType.DMA,
        ],
    )
    def k(t_hbm, i_hbm, o_hbm, iv, rv, sem):
        wid = lax.axis_index("s") * 2 + lax.axis_index("c")
        base = wid * bpw

        def body(i, carry):
            b0 = jnp.minimum(base + i * chunk, base + bpw - chunk)
            pltpu.sync_copy(i_hbm.at[pl.ds(b0, chunk)], iv)
            pltpu.async_copy(t_hbm.at[iv], rv, sem).wait()
            pltpu.sync_copy(rv, o_hbm.at[pl.ds(b0, chunk)])
            return carry

        lax.fori_loop(0, nch, body, 0)

    return k(table, idx)


def _make_msg(has_v):
    scratches = [
        pltpu.VMEM((CE,), jnp.int32),
        pltpu.VMEM((CE,), jnp.int32),
        pltpu.VMEM((16,), jnp.int32),
        pltpu.VMEM((CE, 3 * D), jnp.float32),
        pltpu.VMEM((CE, 3 * D), jnp.float32),
        pltpu.VMEM((CE, 48), jnp.float32),
        pltpu.VMEM((TW, 4 * D), jnp.float32),
        pltpu.SemaphoreType.DMA,
        pltpu.SemaphoreType.DMA,
    ]
    if has_v:
        scratches.insert(4, pltpu.VMEM((CE, 3 * D), jnp.float32))

    def body(refs):
        if has_v:
            (phi_hbm, v_hbm, wfc_hbm, rh_hbm, src_hbm, dst_hbm, rp_hbm,
             out_hbm, sidx, didx, rpv, phib, vb, wb, rhb, acc,
             sem_a, sem_b) = refs
        else:
            (phi_hbm, wfc_hbm, rh_hbm, src_hbm, dst_hbm, rp_hbm,
             out_hbm, sidx, didx, rpv, phib, wb, rhb, acc,
             sem_a, sem_b) = refs
        c = lax.axis_index("c")
        s = lax.axis_index("s")
        wid = s * 2 + c
        lane = lax.broadcasted_iota(jnp.int32, (16,), 0)
        zf = jnp.zeros((16,), jnp.float32)

        def pass_body(p, carry):
            w0 = (p * 32 + wid) * TW

            def zrow(i, cy):
                for k in range(4 * D // 16):
                    acc[i, pl.ds(k * 16, 16)] = zf
                return cy

            lax.fori_loop(0, TW, zrow, 0)

            pltpu.sync_copy(rp_hbm.at[pl.ds(w0, 16)], rpv)
            e0 = jnp.sum(jnp.where(lane == 0, rpv[...], 0))
            pltpu.sync_copy(rp_hbm.at[pl.ds(w0 + TW, 16)], rpv)
            e1 = jnp.sum(jnp.where(lane == 0, rpv[...], 0))
            c0 = e0 // CE
            nch = jnp.maximum(0, (e1 + CE - 1) // CE - c0)

            def chunk(i, cy):
                ec = (c0 + i) * CE
                pltpu.sync_copy(src_hbm.at[pl.ds(ec, CE)], sidx)
                cp_a = pltpu.async_copy(phi_hbm.at[sidx], phib, sem_a)
                if has_v:
                    cp_b = pltpu.async_copy(v_hbm.at[sidx], vb, sem_b)
                pltpu.sync_copy(wfc_hbm.at[pl.ds(ec, CE)], wb)
                pltpu.sync_copy(rh_hbm.at[pl.ds(ec, CE)], rhb)
                pltpu.sync_copy(dst_hbm.at[pl.ds(ec, CE)], didx)
                cp_a.wait()
                if has_v:
                    cp_b.wait()

                def row(j, cy2):
                    h16 = (j // 16) * 16
                    dvals = didx[pl.ds(h16, 16)]
                    r = jnp.sum(jnp.where(lane == j - h16, dvals, 0)) - w0
                    rl = jnp.minimum(jnp.maximum(r, 0), TW - 1)

                    def do_row():
                        rx = rhb[j, pl.ds(0, 16)]
                        ry = rhb[j, pl.ds(16, 16)]
                        rz = rhb[j, pl.ds(32, 16)]
                        for k in range(D // 16):
                            o = k * 16
                            m1 = phib[j, pl.ds(o, 16)] * wb[j, pl.ds(o, 16)]
                            dv1 = (phib[j, pl.ds(D + o, 16)]
                                   * wb[j, pl.ds(D + o, 16)])
                            dv2 = (phib[j, pl.ds(2 * D + o, 16)]
                                   * wb[j, pl.ds(2 * D + o, 16)])
                            acc[rl, pl.ds(o, 16)] = acc[rl, pl.ds(o, 16)] + m1
                            if has_v:
                                acc[rl, pl.ds(D + o, 16)] = (
                                    acc[rl, pl.ds(D + o, 16)]
                                    + dv1 * rx + dv2 * vb[j, pl.ds(o, 16)])
                                acc[rl, pl.ds(2 * D + o, 16)] = (
                                    acc[rl, pl.ds(2 * D + o, 16)]
                                    + dv1 * ry + dv2 * vb[j, pl.ds(D + o, 16)])
                                acc[rl, pl.ds(3 * D + o, 16)] = (
                                    acc[rl, pl.ds(3 * D + o, 16)]
                                    + dv1 * rz
                                    + dv2 * vb[j, pl.ds(2 * D + o, 16)])
                            else:
                                acc[rl, pl.ds(D + o, 16)] = (
                                    acc[rl, pl.ds(D + o, 16)] + dv1 * rx)
                                acc[rl, pl.ds(2 * D + o, 16)] = (
                                    acc[rl, pl.ds(2 * D + o, 16)] + dv1 * ry)
                                acc[rl, pl.ds(3 * D + o, 16)] = (
                                    acc[rl, pl.ds(3 * D + o, 16)] + dv1 * rz)

                    ej = ec + j
                    pl.when((ej >= e0) & (ej < e1))(do_row)
                    return cy2

                lax.fori_loop(0, CE, row, 0)
                return cy

            lax.fori_loop(0, nch, chunk, 0)
            pltpu.sync_copy(acc, out_hbm.at[pl.ds(w0, TW)])
            return carry

        lax.fori_loop(0, PASSES, pass_body, 0)

    def k(*refs):
        body(refs)

    return functools.partial(
        pl.kernel,
        out_type=jax.ShapeDtypeStruct((NOUT, 4 * D), jnp.float32),
        mesh=_mesh,
        scratch_types=scratches,
        compiler_params=pltpu.CompilerParams(needs_layout_passes=False),
    )(k)


_MSG_V = _make_msg(True)
_MSG_NOV = _make_msg(False)


def _silu(x):
    return x * jax.nn.sigmoid(x)


def _tc_geom(ps, pd, W_rbf, b_rbf):
    BE = 1024
    grid = EPAD // BE

    def body(ps_ref, pd_ref, wr_ref, br_ref, wfc0_ref, wfc1_ref, wfc2_ref,
             rh_ref):
        wfc_refs = (wfc0_ref, wfc1_ref, wfc2_ref)
        rij = pd_ref[:, :3] - ps_ref[:, :3]
        d = jnp.sqrt(jnp.sum(rij * rij, axis=1, keepdims=True) + 1e-12)
        d_safe = jnp.maximum(d, 1e-8)
        nfreq = jnp.arange(1, NB + 1, dtype=jnp.int32).astype(
            jnp.float32)[None, :]
        rbf = jnp.sin(nfreq * jnp.pi * d / CUT) / d_safe
        fcut = (0.5 * (jnp.cos(jnp.pi * jnp.clip(d, 0.0, CUT) / CUT) + 1.0)
                * (d < CUT).astype(jnp.float32))
        rhat = rij / d_safe
        rh_ref[...] = jnp.concatenate(
            [jnp.broadcast_to(rhat[:, c:c + 1], (BE, 16)) for c in range(3)],
            axis=1)
        for b in range(NBLK):
            wfc_refs[b][...] = (
                jnp.dot(rbf, wr_ref[b], preferred_element_type=jnp.float32)
                + br_ref[b][None, :]) * fcut

    return pl.pallas_call(
        body,
        grid=(grid,),
        in_specs=[
            pl.BlockSpec((BE, 128), lambda i: (i, 0)),
            pl.BlockSpec((BE, 128), lambda i: (i, 0)),
            pl.BlockSpec((NBLK, NB, 3 * D), lambda i: (0, 0, 0)),
            pl.BlockSpec((NBLK, 3 * D), lambda i: (0, 0)),
        ],
        out_specs=[
            pl.BlockSpec((BE, 3 * D), lambda i: (i, 0)),
            pl.BlockSpec((BE, 3 * D), lambda i: (i, 0)),
            pl.BlockSpec((BE, 3 * D), lambda i: (i, 0)),
            pl.BlockSpec((BE, 48), lambda i: (i, 0)),
        ],
        out_shape=[
            jax.ShapeDtypeStruct((EPAD, 3 * D), jnp.float32),
            jax.ShapeDtypeStruct((EPAD, 3 * D), jnp.float32),
            jax.ShapeDtypeStruct((EPAD, 3 * D), jnp.float32),
            jax.ShapeDtypeStruct((EPAD, 48), jnp.float32),
        ],
    )(ps, pd, W_rbf, b_rbf)


def _tc_phi(x, Wm1, bm1, Wm2, bm2):
    BN = 1024
    grid = NPAD // BN

    def body(x_ref, w1_ref, b1_ref, w2_ref, b2_ref, o_ref):
        h = _silu(jnp.dot(x_ref[...], w1_ref[...],
                          preferred_element_type=jnp.float32) + b1_ref[...])
        o_ref[...] = (jnp.dot(h, w2_ref[...],
                              preferred_element_type=jnp.float32)
                      + b2_ref[...])

    return pl.pallas_call(
        body,
        grid=(grid,),
        in_specs=[
            pl.BlockSpec((BN, D), lambda i: (i, 0)),
            pl.BlockSpec((D, D), lambda i: (0, 0)),
            pl.BlockSpec((1, D), lambda i: (0, 0)),
            pl.BlockSpec((D, 3 * D), lambda i: (0, 0)),
            pl.BlockSpec((1, 3 * D), lambda i: (0, 0)),
        ],
        out_specs=pl.BlockSpec((BN, 3 * D), lambda i: (i, 0)),
        out_shape=jax.ShapeDtypeStruct((NPAD, 3 * D), jnp.float32),
    )(x, Wm1, bm1[None, :], Wm2, bm2[None, :])


def _tc_update(x, v, dd, WU, WV, Wa1, ba1, Wa2, ba2):
    BN = 256
    grid = NPAD // BN

    def body(x_ref, v_ref, dd_ref, wu_ref, wv_ref, wa1_ref, ba1_ref,
             wa2_ref, ba2_ref, xo_ref, vo_ref):
        x1 = x_ref[...] + dd_ref[:, :D]
        v1 = v_ref[...] + dd_ref[:, D:]
        vc = [v1[:, c * D:(c + 1) * D] for c in range(3)]
        U = [jnp.dot(vc[c], wu_ref[...], preferred_element_type=jnp.float32)
             for c in range(3)]
        Vv = [jnp.dot(vc[c], wv_ref[...], preferred_element_type=jnp.float32)
              for c in range(3)]
        Vn = jnp.sqrt(Vv[0] * Vv[0] + Vv[1] * Vv[1] + Vv[2] * Vv[2] + 1e-8)
        h = _silu(jnp.dot(jnp.concatenate([x1, Vn], axis=1), wa1_ref[...],
                          preferred_element_type=jnp.float32) + ba1_ref[...])
        a = (jnp.dot(h, wa2_ref[...], preferred_element_type=jnp.float32)
             + ba2_ref[...])
        a_ss = a[:, :D]
        a_sv = a[:, D:2 * D]
        a_vv = a[:, 2 * D:]
        dot_uv = U[0] * Vv[0] + U[1] * Vv[1] + U[2] * Vv[2]
        xo_ref[...] = x1 + a_ss + a_sv * dot_uv
        vo_ref[...] = jnp.concatenate(
            [vc[c] + a_vv * U[c] for c in range(3)], axis=1)

    return pl.pallas_call(
        body,
        grid=(grid,),
        in_specs=[
            pl.BlockSpec((BN, D), lambda i: (i, 0)),
            pl.BlockSpec((BN, 3 * D), lambda i: (i, 0)),
            pl.BlockSpec((BN, 4 * D), lambda i: (i, 0)),
            pl.BlockSpec((D, D), lambda i: (0, 0)),
            pl.BlockSpec((D, D), lambda i: (0, 0)),
            pl.BlockSpec((2 * D, D), lambda i: (0, 0)),
            pl.BlockSpec((1, D), lambda i: (0, 0)),
            pl.BlockSpec((D, 3 * D), lambda i: (0, 0)),
            pl.BlockSpec((1, 3 * D), lambda i: (0, 0)),
        ],
        out_specs=[
            pl.BlockSpec((BN, D), lambda i: (i, 0)),
            pl.BlockSpec((BN, 3 * D), lambda i: (i, 0)),
        ],
        out_shape=[
            jax.ShapeDtypeStruct((NPAD, D), jnp.float32),
            jax.ShapeDtypeStruct((NPAD, 3 * D), jnp.float32),
        ],
    )(x, v, dd, WU, WV, Wa1, ba1[None, :], Wa2, ba2[None, :])


def _tc_readout(x, onehot, Wo1, bo1, Wo2, bo2):
    BN = 1024
    grid = NPAD // BN

    def body(x_ref, oh_ref, w1_ref, b1_ref, w2_ref, b2_ref, o_ref):
        h = _silu(jnp.dot(x_ref[...], w1_ref[...],
                          preferred_element_type=jnp.float32) + b1_ref[...])
        y = (jnp.dot(h, w2_ref[...], preferred_element_type=jnp.float32)
             + b2_ref[...])
        contrib = jnp.dot(oh_ref[...].T, y,
                          preferred_element_type=jnp.float32)
        @pl.when(pl.program_id(0) == 0)
        def _():
            o_ref[...] = jnp.zeros_like(o_ref)
        o_ref[...] += contrib

    return pl.pallas_call(
        body,
        grid=(grid,),
        in_specs=[
            pl.BlockSpec((BN, D), lambda i: (i, 0)),
            pl.BlockSpec((BN, NG), lambda i: (i, 0)),
            pl.BlockSpec((D, H), lambda i: (0, 0)),
            pl.BlockSpec((1, H), lambda i: (0, 0)),
            pl.BlockSpec((H, 1), lambda i: (0, 0)),
            pl.BlockSpec((1, 1), lambda i: (0, 0)),
        ],
        out_specs=pl.BlockSpec((NG, 1), lambda i: (0, 0)),
        out_shape=jax.ShapeDtypeStruct((NG, 1), jnp.float32),
    )(x, onehot, Wo1, bo1[None, :], Wo2, bo2[None, :])


def kernel(at_no, pos, edge_index, batch, embed, W_rbf, b_rbf, Wm1, bm1,
           Wm2, bm2, WU, WV, Wa1, ba1, Wa2, ba2, Wo1, bo1, Wo2, bo2):
    i32 = jnp.int32
    src = edge_index[0].astype(i32)
    dst = edge_index[1].astype(i32)
    perm = jnp.argsort(dst)
    src_s = src[perm]
    dst_s = dst[perm]
    src_p = jnp.concatenate([src_s, jnp.zeros((EPAD - E,), i32)])
    dst_p = jnp.concatenate([dst_s, jnp.zeros((EPAD - E,), i32)])
    rp = jnp.searchsorted(dst_s, jnp.arange(RPLEN, dtype=i32)).astype(i32)
    at_p = jnp.concatenate([at_no.astype(i32), jnp.zeros((NPAD - N,), i32)])
    bt = jnp.concatenate([batch.astype(i32), jnp.full((NPAD - N,), NG, i32)])
    onehot = (bt[:, None] == jnp.arange(NG, dtype=i32)[None, :]).astype(
        jnp.float32)
    pos128 = jnp.pad(pos, ((0, 0), (0, 125)))

    x = _sc_gather(embed, at_p, 320)
    ps = _sc_gather(pos128, src_p, 512)
    pd = _sc_gather(pos128, dst_p, 512)
    wfc0, wfc1, wfc2, rh = _tc_geom(ps, pd, W_rbf, b_rbf)
    wfcs = (wfc0, wfc1, wfc2)

    v = jnp.zeros((NPAD, 3 * D), jnp.float32)
    for b in range(NBLK):
        phi = _tc_phi(x, Wm1[b], bm1[b], Wm2[b], bm2[b])
        if b == 0:
            dd = _MSG_NOV(phi, wfcs[b], rh, src_p, dst_p, rp)
        else:
            dd = _MSG_V(phi, v, wfcs[b], rh, src_p, dst_p, rp)
        x, v = _tc_update(x, v, dd, WU[b], WV[b], Wa1[b], ba1[b],
                          Wa2[b], ba2[b])
    return _tc_readout(x, onehot, Wo1, bo1, Wo2, bo2)


# speedup vs baseline: 8.3579x; 1.3426x over previous
import functools

import jax
import jax.numpy as jnp
from jax import lax
from jax.experimental import pallas as pl
from jax.experimental.pallas import tpu as pltpu
from jax.experimental.pallas import tpu_sc as plsc

N = 10000
E = 160000
D = 256
NB = 20
NBLK = 3
NG = 16
H = 512
CUT = 5.0

NPAD = 10240
EPAD = 160768
TW = 40
PASSES = 8
NOUT = 32 * TW * PASSES
RPLEN = NOUT + 16
CE = 16

_mesh = plsc.VectorSubcoreMesh(
    core_axis_name="c", subcore_axis_name="s", num_cores=2, num_subcores=16)


def _sc_gather(table, idx, chunk):
    B = idx.shape[0]
    R = table.shape[1]
    bpw = B // 32
    nch = -(-bpw // chunk)

    @functools.partial(
        pl.kernel,
        out_type=jax.ShapeDtypeStruct((B, R), jnp.float32),
        mesh=_mesh,
        scratch_types=[
            pltpu.VMEM((chunk,), jnp.int32),
            pltpu.VMEM((chunk, R), jnp.float32),
            pltpu.SemaphoreType.DMA,
        ],
    )
    def k(t_hbm, i_hbm, o_hbm, iv, rv, sem):
        wid = lax.axis_index("s") * 2 + lax.axis_index("c")
        base = wid * bpw

        def body(i, carry):
            b0 = jnp.minimum(base + i * chunk, base + bpw - chunk)
            pltpu.sync_copy(i_hbm.at[pl.ds(b0, chunk)], iv)
            pltpu.async_copy(t_hbm.at[iv], rv, sem).wait()
            pltpu.sync_copy(rv, o_hbm.at[pl.ds(b0, chunk)])
            return carry

        lax.fori_loop(0, nch, body, 0)

    return k(table, idx)


def _make_msg(has_v):
    scratches = [
        pltpu.VMEM((2, CE), jnp.int32),
        pltpu.VMEM((2, CE), jnp.int32),
        pltpu.VMEM((16,), jnp.int32),
        pltpu.VMEM((2, CE, 3 * D), jnp.float32),
        pltpu.VMEM((2, CE, 3 * D), jnp.float32),
        pltpu.VMEM((2, CE, 48), jnp.float32),
        pltpu.VMEM((TW, 4 * D), jnp.float32),
        pltpu.SemaphoreType.DMA,
        pltpu.SemaphoreType.DMA,
    ]
    if has_v:
        scratches.insert(4, pltpu.VMEM((2, CE, 3 * D), jnp.float32))

    def body(refs):
        if has_v:
            (phi_hbm, v_hbm, wfc_hbm, rh_hbm, src_hbm, dst_hbm, rp_hbm,
             out_hbm, sidx, didx, rpv, phib, vb, wb, rhb, acc,
             sem_a, sem_b) = refs
        else:
            (phi_hbm, wfc_hbm, rh_hbm, src_hbm, dst_hbm, rp_hbm,
             out_hbm, sidx, didx, rpv, phib, wb, rhb, acc,
             sem_a, sem_b) = refs
        c = lax.axis_index("c")
        s = lax.axis_index("s")
        wid = s * 2 + c
        lane = lax.broadcasted_iota(jnp.int32, (16,), 0)
        zf = jnp.zeros((16,), jnp.float32)

        def pass_body(p, carry):
            w0 = (p * 32 + wid) * TW

            def zrow(i, cy):
                for k in range(4 * D // 16):
                    acc[i, pl.ds(k * 16, 16)] = zf
                return cy

            lax.fori_loop(0, TW, zrow, 0)

            pltpu.sync_copy(rp_hbm.at[pl.ds(w0, 16)], rpv)
            e0 = jnp.sum(jnp.where(lane == 0, rpv[...], 0))
            pltpu.sync_copy(rp_hbm.at[pl.ds(w0 + TW, 16)], rpv)
            e1 = jnp.sum(jnp.where(lane == 0, rpv[...], 0))
            c0 = e0 // CE
            nch = jnp.maximum(0, (e1 + CE - 1) // CE - c0)
            def issue(i, slot):
                ec = (c0 + i) * CE

                def go():
                    pltpu.sync_copy(src_hbm.at[pl.ds(ec, CE)], sidx.at[slot])
                    pltpu.async_copy(dst_hbm.at[pl.ds(ec, CE)],
                                     didx.at[slot], sem_a)
                    pltpu.async_copy(phi_hbm.at[sidx.at[slot]],
                                     phib.at[slot], sem_a)
                    if has_v:
                        pltpu.async_copy(v_hbm.at[sidx.at[slot]],
                                         vb.at[slot], sem_a)
                    pltpu.async_copy(wfc_hbm.at[pl.ds(ec, CE)],
                                     wb.at[slot], sem_a)
                    pltpu.async_copy(rh_hbm.at[pl.ds(ec, CE)],
                                     rhb.at[slot], sem_a)
                return go

            def drain(i, slot):
                ec = (c0 + i) * CE
                pltpu.make_async_copy(dst_hbm.at[pl.ds(ec, CE)],
                                      didx.at[slot], sem_a).wait()
                pltpu.make_async_copy(phi_hbm.at[sidx.at[slot]],
                                      phib.at[slot], sem_a).wait()
                if has_v:
                    pltpu.make_async_copy(v_hbm.at[sidx.at[slot]],
                                          vb.at[slot], sem_a).wait()
                pltpu.make_async_copy(wfc_hbm.at[pl.ds(ec, CE)],
                                      wb.at[slot], sem_a).wait()
                pltpu.make_async_copy(rh_hbm.at[pl.ds(ec, CE)],
                                      rhb.at[slot], sem_a).wait()

            pl.when(nch > 0)(issue(0, 0))

            def chunk(i, cy):
                slot = i % 2
                ec = (c0 + i) * CE
                drain(i, slot)
                pl.when(i + 1 < nch)(issue(i + 1, (i + 1) % 2))

                def row(j, cy2):
                    dvals = didx[slot, pl.ds(0, 16)]
                    r = jnp.sum(jnp.where(lane == j, dvals, 0)) - w0
                    rl = jnp.minimum(jnp.maximum(r, 0), TW - 1)

                    def do_row():
                        rx = rhb[slot, j, pl.ds(0, 16)]
                        ry = rhb[slot, j, pl.ds(16, 16)]
                        rz = rhb[slot, j, pl.ds(32, 16)]
                        for k in range(D // 16):
                            o = k * 16
                            m1 = (phib[slot, j, pl.ds(o, 16)]
                                  * wb[slot, j, pl.ds(o, 16)])
                            dv1 = (phib[slot, j, pl.ds(D + o, 16)]
                                   * wb[slot, j, pl.ds(D + o, 16)])
                            dv2 = (phib[slot, j, pl.ds(2 * D + o, 16)]
                                   * wb[slot, j, pl.ds(2 * D + o, 16)])
                            plsc.addupdate(acc.at[rl, pl.ds(o, 16)], m1)
                            if has_v:
                                plsc.addupdate(
                                    acc.at[rl, pl.ds(D + o, 16)],
                                    dv1 * rx + dv2 * vb[slot, j, pl.ds(o, 16)])
                                plsc.addupdate(
                                    acc.at[rl, pl.ds(2 * D + o, 16)],
                                    dv1 * ry
                                    + dv2 * vb[slot, j, pl.ds(D + o, 16)])
                                plsc.addupdate(
                                    acc.at[rl, pl.ds(3 * D + o, 16)],
                                    dv1 * rz
                                    + dv2 * vb[slot, j, pl.ds(2 * D + o, 16)])
                            else:
                                plsc.addupdate(
                                    acc.at[rl, pl.ds(D + o, 16)], dv1 * rx)
                                plsc.addupdate(
                                    acc.at[rl, pl.ds(2 * D + o, 16)], dv1 * ry)
                                plsc.addupdate(
                                    acc.at[rl, pl.ds(3 * D + o, 16)], dv1 * rz)

                    ej = ec + j
                    pl.when((ej >= e0) & (ej < e1))(do_row)
                    return cy2

                lax.fori_loop(0, CE, row, 0)
                return cy

            lax.fori_loop(0, nch, chunk, 0)
            pltpu.sync_copy(acc, out_hbm.at[pl.ds(w0, TW)])
            return carry

        lax.fori_loop(0, PASSES, pass_body, 0)

    def k(*refs):
        body(refs)

    return functools.partial(
        pl.kernel,
        out_type=jax.ShapeDtypeStruct((NOUT, 4 * D), jnp.float32),
        mesh=_mesh,
        scratch_types=scratches,
        compiler_params=pltpu.CompilerParams(needs_layout_passes=False),
    )(k)


_MSG_V = _make_msg(True)
_MSG_NOV = _make_msg(False)


def _silu(x):
    return x * jax.nn.sigmoid(x)


def _tc_geom(ps, pd, W_rbf, b_rbf):
    BE = 1024
    grid = EPAD // BE

    def body(ps_ref, pd_ref, wr_ref, br_ref, wfc0_ref, wfc1_ref, wfc2_ref,
             rh_ref):
        wfc_refs = (wfc0_ref, wfc1_ref, wfc2_ref)
        rij = pd_ref[:, :3] - ps_ref[:, :3]
        d = jnp.sqrt(jnp.sum(rij * rij, axis=1, keepdims=True) + 1e-12)
        d_safe = jnp.maximum(d, 1e-8)
        nfreq = jnp.arange(1, NB + 1, dtype=jnp.int32).astype(
            jnp.float32)[None, :]
        rbf = jnp.sin(nfreq * jnp.pi * d / CUT) / d_safe
        fcut = (0.5 * (jnp.cos(jnp.pi * jnp.clip(d, 0.0, CUT) / CUT) + 1.0)
                * (d < CUT).astype(jnp.float32))
        rhat = rij / d_safe
        rh_ref[...] = jnp.concatenate(
            [jnp.broadcast_to(rhat[:, c:c + 1], (BE, 16)) for c in range(3)],
            axis=1)
        for b in range(NBLK):
            wfc_refs[b][...] = (
                jnp.dot(rbf, wr_ref[b], preferred_element_type=jnp.float32)
                + br_ref[b][None, :]) * fcut

    return pl.pallas_call(
        body,
        grid=(grid,),
        in_specs=[
            pl.BlockSpec((BE, 128), lambda i: (i, 0)),
            pl.BlockSpec((BE, 128), lambda i: (i, 0)),
            pl.BlockSpec((NBLK, NB, 3 * D), lambda i: (0, 0, 0)),
            pl.BlockSpec((NBLK, 3 * D), lambda i: (0, 0)),
        ],
        out_specs=[
            pl.BlockSpec((BE, 3 * D), lambda i: (i, 0)),
            pl.BlockSpec((BE, 3 * D), lambda i: (i, 0)),
            pl.BlockSpec((BE, 3 * D), lambda i: (i, 0)),
            pl.BlockSpec((BE, 48), lambda i: (i, 0)),
        ],
        out_shape=[
            jax.ShapeDtypeStruct((EPAD, 3 * D), jnp.float32),
            jax.ShapeDtypeStruct((EPAD, 3 * D), jnp.float32),
            jax.ShapeDtypeStruct((EPAD, 3 * D), jnp.float32),
            jax.ShapeDtypeStruct((EPAD, 48), jnp.float32),
        ],
    )(ps, pd, W_rbf, b_rbf)


def _tc_phi(x, Wm1, bm1, Wm2, bm2):
    BN = 1024
    grid = NPAD // BN

    def body(x_ref, w1_ref, b1_ref, w2_ref, b2_ref, o_ref):
        h = _silu(jnp.dot(x_ref[...], w1_ref[...],
                          preferred_element_type=jnp.float32) + b1_ref[...])
        o_ref[...] = (jnp.dot(h, w2_ref[...],
                              preferred_element_type=jnp.float32)
                      + b2_ref[...])

    return pl.pallas_call(
        body,
        grid=(grid,),
        in_specs=[
            pl.BlockSpec((BN, D), lambda i: (i, 0)),
            pl.BlockSpec((D, D), lambda i: (0, 0)),
            pl.BlockSpec((1, D), lambda i: (0, 0)),
            pl.BlockSpec((D, 3 * D), lambda i: (0, 0)),
            pl.BlockSpec((1, 3 * D), lambda i: (0, 0)),
        ],
        out_specs=pl.BlockSpec((BN, 3 * D), lambda i: (i, 0)),
        out_shape=jax.ShapeDtypeStruct((NPAD, 3 * D), jnp.float32),
    )(x, Wm1, bm1[None, :], Wm2, bm2[None, :])


def _tc_update(x, v, dd, WU, WV, Wa1, ba1, Wa2, ba2):
    BN = 256
    grid = NPAD // BN

    def body(x_ref, v_ref, dd_ref, wu_ref, wv_ref, wa1_ref, ba1_ref,
             wa2_ref, ba2_ref, xo_ref, vo_ref):
        x1 = x_ref[...] + dd_ref[:, :D]
        v1 = v_ref[...] + dd_ref[:, D:]
        vc = [v1[:, c * D:(c + 1) * D] for c in range(3)]
        U = [jnp.dot(vc[c], wu_ref[...], preferred_element_type=jnp.float32)
             for c in range(3)]
        Vv = [jnp.dot(vc[c], wv_ref[...], preferred_element_type=jnp.float32)
              for c in range(3)]
        Vn = jnp.sqrt(Vv[0] * Vv[0] + Vv[1] * Vv[1] + Vv[2] * Vv[2] + 1e-8)
        h = _silu(jnp.dot(jnp.concatenate([x1, Vn], axis=1), wa1_ref[...],
                          preferred_element_type=jnp.float32) + ba1_ref[...])
        a = (jnp.dot(h, wa2_ref[...], preferred_element_type=jnp.float32)
             + ba2_ref[...])
        a_ss = a[:, :D]
        a_sv = a[:, D:2 * D]
        a_vv = a[:, 2 * D:]
        dot_uv = U[0] * Vv[0] + U[1] * Vv[1] + U[2] * Vv[2]
        xo_ref[...] = x1 + a_ss + a_sv * dot_uv
        vo_ref[...] = jnp.concatenate(
            [vc[c] + a_vv * U[c] for c in range(3)], axis=1)

    return pl.pallas_call(
        body,
        grid=(grid,),
        in_specs=[
            pl.BlockSpec((BN, D), lambda i: (i, 0)),
            pl.BlockSpec((BN, 3 * D), lambda i: (i, 0)),
            pl.BlockSpec((BN, 4 * D), lambda i: (i, 0)),
            pl.BlockSpec((D, D), lambda i: (0, 0)),
            pl.BlockSpec((D, D), lambda i: (0, 0)),
            pl.BlockSpec((2 * D, D), lambda i: (0, 0)),
            pl.BlockSpec((1, D), lambda i: (0, 0)),
            pl.BlockSpec((D, 3 * D), lambda i: (0, 0)),
            pl.BlockSpec((1, 3 * D), lambda i: (0, 0)),
        ],
        out_specs=[
            pl.BlockSpec((BN, D), lambda i: (i, 0)),
            pl.BlockSpec((BN, 3 * D), lambda i: (i, 0)),
        ],
        out_shape=[
            jax.ShapeDtypeStruct((NPAD, D), jnp.float32),
            jax.ShapeDtypeStruct((NPAD, 3 * D), jnp.float32),
        ],
    )(x, v, dd, WU, WV, Wa1, ba1[None, :], Wa2, ba2[None, :])


def _tc_readout(x, onehot, Wo1, bo1, Wo2, bo2):
    BN = 1024
    grid = NPAD // BN

    def body(x_ref, oh_ref, w1_ref, b1_ref, w2_ref, b2_ref, o_ref):
        h = _silu(jnp.dot(x_ref[...], w1_ref[...],
                          preferred_element_type=jnp.float32) + b1_ref[...])
        y = (jnp.dot(h, w2_ref[...], preferred_element_type=jnp.float32)
             + b2_ref[...])
        contrib = jnp.dot(oh_ref[...].T, y,
                          preferred_element_type=jnp.float32)
        @pl.when(pl.program_id(0) == 0)
        def _():
            o_ref[...] = jnp.zeros_like(o_ref)
        o_ref[...] += contrib

    return pl.pallas_call(
        body,
        grid=(grid,),
        in_specs=[
            pl.BlockSpec((BN, D), lambda i: (i, 0)),
            pl.BlockSpec((BN, NG), lambda i: (i, 0)),
            pl.BlockSpec((D, H), lambda i: (0, 0)),
            pl.BlockSpec((1, H), lambda i: (0, 0)),
            pl.BlockSpec((H, 1), lambda i: (0, 0)),
            pl.BlockSpec((1, 1), lambda i: (0, 0)),
        ],
        out_specs=pl.BlockSpec((NG, 1), lambda i: (0, 0)),
        out_shape=jax.ShapeDtypeStruct((NG, 1), jnp.float32),
    )(x, onehot, Wo1, bo1[None, :], Wo2, bo2[None, :])


def kernel(at_no, pos, edge_index, batch, embed, W_rbf, b_rbf, Wm1, bm1,
           Wm2, bm2, WU, WV, Wa1, ba1, Wa2, ba2, Wo1, bo1, Wo2, bo2):
    i32 = jnp.int32
    src = edge_index[0].astype(i32)
    dst = edge_index[1].astype(i32)
    key = jnp.sort(dst * 16384 + src)
    src_s = key & 16383
    dst_s = key >> 14
    src_p = jnp.concatenate([src_s, jnp.zeros((EPAD - E,), i32)])
    dst_p = jnp.concatenate([dst_s, jnp.zeros((EPAD - E,), i32)])
    rp = jnp.searchsorted(dst_s, jnp.arange(RPLEN, dtype=i32)).astype(i32)
    at_p = jnp.concatenate([at_no.astype(i32), jnp.zeros((NPAD - N,), i32)])
    bt = jnp.concatenate([batch.astype(i32), jnp.full((NPAD - N,), NG, i32)])
    onehot = (bt[:, None] == jnp.arange(NG, dtype=i32)[None, :]).astype(
        jnp.float32)
    pos128 = jnp.pad(pos, ((0, 0), (0, 125)))

    x = _sc_gather(embed, at_p, 320)
    ps = _sc_gather(pos128, src_p, 512)
    pd = _sc_gather(pos128, dst_p, 512)
    wfc0, wfc1, wfc2, rh = _tc_geom(ps, pd, W_rbf, b_rbf)
    wfcs = (wfc0, wfc1, wfc2)

    v = jnp.zeros((NPAD, 3 * D), jnp.float32)
    for b in range(NBLK):
        phi = _tc_phi(x, Wm1[b], bm1[b], Wm2[b], bm2[b])
        if b == 0:
            dd = _MSG_NOV(phi, wfcs[b], rh, src_p, dst_p, rp)
        else:
            dd = _MSG_V(phi, v, wfcs[b], rh, src_p, dst_p, rp)
        x, v = _tc_update(x, v, dd, WU[b], WV[b], Wa1[b], ba1[b],
                          Wa2[b], ba2[b])
    return _tc_readout(x, onehot, Wo1, bo1, Wo2, bo2)
